# Initial kernel scaffold; baseline (speedup 1.0000x reference)
import jax
import jax.numpy as jnp
from jax.experimental import pallas as pl


def kernel(x, edge_index, batch, W_proj, b_proj, Wg, W_ih, b_ih, W_hh, b_hh, W1, b1, W2, b2):
    raise NotImplementedError("write your pallas kernel here")


# speedup vs baseline: 2.7540x; 2.7540x over previous
import functools

import jax
import jax.numpy as jnp
from jax import lax
from jax.experimental import pallas as pl
from jax.experimental.pallas import tpu as pltpu
from jax.experimental.pallas import tpu_sc as plsc

_N = 10000
_E = 320000
_IN = 128
_H = 200
_HA = 128
_HB = _H - _HA
_STEPS = 8
_G = 64
_HID = 100
_OUT = 2

_NC = 2
_NS = 16
_CH = 80
_EPT = _E // _NS
_ITERS = _EPT // _CH
_RPS = 624
_TAIL = _N - _NS * _RPS

_BLK = 1000
_NBLK = _N // _BLK


def _sc_segment_sum(m_a, m_b, src3d, dst3d, zrows):
    mesh = plsc.VectorSubcoreMesh(core_axis_name="c", subcore_axis_name="s")

    @functools.partial(
        pl.kernel,
        mesh=mesh,
        out_type=jax.ShapeDtypeStruct((_NC * _N, _HA), jnp.float32),
        scratch_types=[
            pltpu.VMEM((1, 1, _CH), jnp.int32),
            pltpu.VMEM((1, 1, _CH), jnp.int32),
            pltpu.VMEM((_CH, _HA), jnp.float32),
            pltpu.VMEM_SHARED((_N, _HA), jnp.float32),
            pltpu.SemaphoreType.DMA,
        ],
    )
    def seg_sum(ma_hbm, mb_hbm, src_hbm, dst_hbm, z_hbm, out_hbm, sidx, didx,
                rows, acc, sem):
        cid = lax.axis_index("c")
        sid = lax.axis_index("s")
        pltpu.sync_copy(z_hbm.at[pl.ds(0, _RPS)],
                        acc.at[pl.ds(sid * _RPS, _RPS)])

        @pl.when(sid == _NS - 1)
        def _():
            pltpu.sync_copy(z_hbm.at[pl.ds(0, _TAIL)],
                            acc.at[pl.ds(_NS * _RPS, _TAIL)])

        plsc.subcore_barrier()

        def _accumulate(m_hbm):
            def body(j, carry):
                pltpu.sync_copy(src_hbm.at[pl.ds(sid, 1), pl.ds(j, 1)], sidx)
                pltpu.sync_copy(dst_hbm.at[pl.ds(sid, 1), pl.ds(j, 1)], didx)
                pltpu.async_copy(m_hbm.at[sidx.at[0, 0]], rows, sem).wait()
                pltpu.sync_copy(rows, acc.at[didx.at[0, 0]], add=True)
                return carry
            lax.fori_loop(0, _ITERS, body, 0)

        @pl.when(cid == 0)
        def _():
            _accumulate(ma_hbm)

        @pl.when(cid == 1)
        def _():
            _accumulate(mb_hbm)

        plsc.subcore_barrier()
        pltpu.sync_copy(
            acc.at[pl.ds(sid * _RPS, _RPS)],
            out_hbm.at[pl.ds(cid * _N + sid * _RPS, _RPS)])

        @pl.when(sid == _NS - 1)
        def _():
            pltpu.sync_copy(
                acc.at[pl.ds(_NS * _RPS, _TAIL)],
                out_hbm.at[pl.ds(cid * _N + _NS * _RPS, _TAIL)])

    return seg_sum(m_a, m_b, src3d, dst3d, zrows)


def _dot(a, b):
    return jnp.dot(a, b, preferred_element_type=jnp.float32)


def _proj_body(x_ref, wp_ref, bp_ref, wga_ref, wgb_ref, h_ref, ma_ref,
               mb_ref):
    h = jnp.maximum(_dot(x_ref[...], wp_ref[...]) + bp_ref[...], 0.0)
    h_ref[...] = h
    ma_ref[...] = _dot(h, wga_ref[...])
    mb_ref[...] = _dot(h, wgb_ref[...])


def _proj(x, wpT, bp, wgA, wgB):
    return pl.pallas_call(
        _proj_body,
        grid=(_NBLK,),
        in_specs=[
            pl.BlockSpec((_BLK, _IN), lambda i: (i, 0)),
            pl.BlockSpec((_IN, _H), lambda i: (0, 0)),
            pl.BlockSpec((1, _H), lambda i: (0, 0)),
            pl.BlockSpec((_H, _HA), lambda i: (0, 0)),
            pl.BlockSpec((_H, _HA), lambda i: (0, 0)),
        ],
        out_specs=[
            pl.BlockSpec((_BLK, _H), lambda i: (i, 0)),
            pl.BlockSpec((_BLK, _HA), lambda i: (i, 0)),
            pl.BlockSpec((_BLK, _HA), lambda i: (i, 0)),
        ],
        out_shape=[
            jax.ShapeDtypeStruct((_N, _H), jnp.float32),
            jax.ShapeDtypeStruct((_N, _HA), jnp.float32),
            jax.ShapeDtypeStruct((_N, _HA), jnp.float32),
        ],
    )(x, wpT, bp, wgA, wgB)


def _gru_compute(aa, ab, h, wm, bias):
    (wira, wirb, wiza, wizb, wina, winb, whr, whz, whn) = wm
    r = jax.nn.sigmoid(_dot(aa, wira) + _dot(ab, wirb) + _dot(h, whr)
                       + bias[0:1, :])
    z = jax.nn.sigmoid(_dot(aa, wiza) + _dot(ab, wizb) + _dot(h, whz)
                       + bias[1:2, :])
    n = jnp.tanh(_dot(aa, wina) + _dot(ab, winb) + bias[2:3, :]
                 + r * (_dot(h, whn) + bias[3:4, :]))
    return (1.0 - z) * n + z * h


def _gru_next_body(aa_ref, ab_ref, h_ref, w0, w1, w2, w3, w4, w5, w6, w7, w8,
                   bias_ref, wga_ref, wgb_ref, hn_ref, ma_ref, mb_ref):
    wm = tuple(w[...] for w in (w0, w1, w2, w3, w4, w5, w6, w7, w8))
    h_new = _gru_compute(aa_ref[...], ab_ref[...], h_ref[...], wm, bias_ref[...])
    hn_ref[...] = h_new
    ma_ref[...] = _dot(h_new, wga_ref[...])
    mb_ref[...] = _dot(h_new, wgb_ref[...])


def _gru_last_body(aa_ref, ab_ref, h_ref, w0, w1, w2, w3, w4, w5, w6, w7, w8,
                   bias_ref, hn_ref):
    wm = tuple(w[...] for w in (w0, w1, w2, w3, w4, w5, w6, w7, w8))
    hn_ref[...] = _gru_compute(aa_ref[...], ab_ref[...], h_ref[...], wm,
                               bias_ref[...])


def _gru_step(parts, h, wmats, bias, wg_next):
    wspecs = ([pl.BlockSpec((_HA, _H), lambda i: (0, 0))] * 6
              + [pl.BlockSpec((_H, _H), lambda i: (0, 0))] * 3)
    in_specs = [
        pl.BlockSpec((_BLK, _HA), lambda i: (i, 0)),
        pl.BlockSpec((_BLK, _HA), lambda i: (i + _NBLK, 0)),
        pl.BlockSpec((_BLK, _H), lambda i: (i, 0)),
    ] + wspecs + [pl.BlockSpec((8, _H), lambda i: (0, 0))]
    if wg_next is not None:
        return pl.pallas_call(
            _gru_next_body,
            grid=(_NBLK,),
            in_specs=in_specs + [
                pl.BlockSpec((_H, _HA), lambda i: (0, 0)),
                pl.BlockSpec((_H, _HA), lambda i: (0, 0)),
            ],
            out_specs=[
                pl.BlockSpec((_BLK, _H), lambda i: (i, 0)),
                pl.BlockSpec((_BLK, _HA), lambda i: (i, 0)),
                pl.BlockSpec((_BLK, _HA), lambda i: (i, 0)),
            ],
            out_shape=[
                jax.ShapeDtypeStruct((_N, _H), jnp.float32),
                jax.ShapeDtypeStruct((_N, _HA), jnp.float32),
                jax.ShapeDtypeStruct((_N, _HA), jnp.float32),
            ],
        )(parts, parts, h, *wmats, bias, *wg_next)
    return pl.pallas_call(
        _gru_last_body,
        grid=(_NBLK,),
        in_specs=in_specs,
        out_specs=pl.BlockSpec((_BLK, _H), lambda i: (i, 0)),
        out_shape=jax.ShapeDtypeStruct((_N, _H), jnp.float32),
    )(parts, parts, h, *wmats, bias)


def _pool_mlp_body(h_ref, b_ref, w1_ref, b1_ref, w2_ref, b2_ref, out_ref,
                   acc_ref):
    i = pl.program_id(0)
    hb = h_ref[...]
    bb = b_ref[...]
    neg = jnp.float32(-jnp.inf)
    local = jnp.stack(
        [jnp.max(jnp.where(bb == g, hb, neg), axis=0) for g in range(_G)])

    @pl.when(i == 0)
    def _():
        acc_ref[...] = local

    @pl.when(i > 0)
    def _():
        acc_ref[...] = jnp.maximum(acc_ref[...], local)

    @pl.when(i == _NBLK - 1)
    def _():
        pooled = jnp.maximum(acc_ref[...], 0.0)
        hid = jnp.maximum(_dot(pooled, w1_ref[...]) + b1_ref[...], 0.0)
        out_ref[...] = _dot(hid, w2_ref[...]) + b2_ref[...]


def _pool_mlp(h, batch2d, w1T, b1, w2T, b2):
    return pl.pallas_call(
        _pool_mlp_body,
        grid=(_NBLK,),
        in_specs=[
            pl.BlockSpec((_BLK, _H), lambda i: (i, 0)),
            pl.BlockSpec((_BLK, 1), lambda i: (i, 0)),
            pl.BlockSpec((_H, _HID), lambda i: (0, 0)),
            pl.BlockSpec((1, _HID), lambda i: (0, 0)),
            pl.BlockSpec((_HID, _OUT), lambda i: (0, 0)),
            pl.BlockSpec((1, _OUT), lambda i: (0, 0)),
        ],
        out_specs=pl.BlockSpec((_G, _OUT), lambda i: (0, 0)),
        out_shape=jax.ShapeDtypeStruct((_G, _OUT), jnp.float32),
        scratch_shapes=[pltpu.VMEM((_G, _H), jnp.float32)],
    )(h, batch2d, w1T, b1, w2T, b2)


def _split_k(wT):
    top = wT[:_HA, :]
    bot = jnp.concatenate(
        [wT[_HA:, :], jnp.zeros((_HA - _HB, _H), jnp.float32)], axis=0)
    return top, bot


def _split_wg(wg):
    wga = wg[:, :_HA]
    wgb = jnp.concatenate(
        [wg[:, _HA:], jnp.zeros((_H, _HA - _HB), jnp.float32)], axis=1)
    return wga, wgb


def kernel(x, edge_index, batch, W_proj, b_proj, Wg, W_ih, b_ih, W_hh, b_hh,
           W1, b1, W2, b2):
    src3d = edge_index[0].reshape(_NS, _ITERS, _CH)
    dst3d = edge_index[1].reshape(_NS, _ITERS, _CH)
    zrows = jnp.zeros((_RPS, _HA), jnp.float32)

    wi = jnp.split(W_ih, 3, axis=0)
    wh = jnp.split(W_hh, 3, axis=0)
    wmats = (_split_k(wi[0].T) + _split_k(wi[1].T) + _split_k(wi[2].T)
             + (wh[0].T, wh[1].T, wh[2].T))
    bi = jnp.split(b_ih, 3)
    bh = jnp.split(b_hh, 3)
    bias = jnp.concatenate(
        [jnp.stack([bi[0] + bh[0], bi[1] + bh[1], bi[2], bh[2]]),
         jnp.zeros((4, _H), jnp.float32)])

    wg0a, wg0b = _split_wg(Wg[0])
    h, m_a, m_b = _proj(x, W_proj.T, b_proj[None, :], wg0a, wg0b)
    for i in range(_STEPS):
        parts = _sc_segment_sum(m_a, m_b, src3d, dst3d, zrows)
        if i + 1 < _STEPS:
            h, m_a, m_b = _gru_step(parts, h, wmats, bias, _split_wg(Wg[i + 1]))
        else:
            h = _gru_step(parts, h, wmats, bias, None)

    return _pool_mlp(h, batch[:, None], W1.T, b1[None, :], W2.T, b2[None, :])


# speedup vs baseline: 4.8603x; 1.7648x over previous
import functools

import jax
import jax.numpy as jnp
from jax import lax
from jax.experimental import pallas as pl
from jax.experimental.pallas import tpu as pltpu
from jax.experimental.pallas import tpu_sc as plsc

_N = 10000
_E = 320000
_IN = 128
_H = 200
_HA = 128
_HB = _H - _HA
_STEPS = 8
_G = 64
_HID = 100
_OUT = 2

_NC = 2
_NS = 16
_CH = 80
_EPT = _E // _NS
_ITERS = _EPT // _CH
_GITERS = 10
_GROUPS = _ITERS // _GITERS
_RPS = 624
_TAIL = _N - _NS * _RPS

_BLK = 1000
_NBLK = _N // _BLK


def _sc_segment_sum(m_a, m_b, src3d, dst3d, zrows):
    mesh = plsc.VectorSubcoreMesh(core_axis_name="c", subcore_axis_name="s")

    @functools.partial(
        pl.kernel,
        mesh=mesh,
        out_type=jax.ShapeDtypeStruct((_NC * _N, _HA), jnp.float32),
        scratch_types=[
            pltpu.VMEM((1, 1, _GITERS, _CH), jnp.int32),
            pltpu.VMEM((1, 1, _GITERS, _CH), jnp.int32),
            pltpu.VMEM((_CH, _HA), jnp.float32),
            pltpu.VMEM((_CH, _HA), jnp.float32),
            pltpu.VMEM_SHARED((_N, _HA), jnp.float32),
            pltpu.SemaphoreType.DMA,
            pltpu.SemaphoreType.DMA,
        ],
    )
    def seg_sum(ma_hbm, mb_hbm, src_hbm, dst_hbm, z_hbm, out_hbm, sidx, didx,
                rows_a, rows_b, acc, sem_a, sem_b):
        cid = lax.axis_index("c")
        sid = lax.axis_index("s")
        pltpu.sync_copy(z_hbm.at[pl.ds(0, _RPS)],
                        acc.at[pl.ds(sid * _RPS, _RPS)])

        @pl.when(sid == _NS - 1)
        def _():
            pltpu.sync_copy(z_hbm.at[pl.ds(0, _TAIL)],
                            acc.at[pl.ds(_NS * _RPS, _TAIL)])

        plsc.subcore_barrier()

        def _accumulate(m_hbm):
            def group(g, carry):
                pltpu.sync_copy(src_hbm.at[pl.ds(sid, 1), pl.ds(g, 1)], sidx)
                pltpu.sync_copy(dst_hbm.at[pl.ds(sid, 1), pl.ds(g, 1)], didx)

                def pair(t, c2):
                    ga = pltpu.async_copy(
                        m_hbm.at[sidx.at[0, 0, 2 * t]], rows_a, sem_a)
                    gb = pltpu.async_copy(
                        m_hbm.at[sidx.at[0, 0, 2 * t + 1]], rows_b, sem_b)
                    ga.wait()
                    pltpu.sync_copy(rows_a, acc.at[didx.at[0, 0, 2 * t]],
                                    add=True)
                    gb.wait()
                    pltpu.sync_copy(rows_b, acc.at[didx.at[0, 0, 2 * t + 1]],
                                    add=True)
                    return c2

                lax.fori_loop(0, _GITERS // 2, pair, 0)
                return carry

            lax.fori_loop(0, _GROUPS, group, 0)

        @pl.when(cid == 0)
        def _():
            _accumulate(ma_hbm)

        @pl.when(cid == 1)
        def _():
            _accumulate(mb_hbm)

        plsc.subcore_barrier()
        pltpu.sync_copy(
            acc.at[pl.ds(sid * _RPS, _RPS)],
            out_hbm.at[pl.ds(cid * _N + sid * _RPS, _RPS)])

        @pl.when(sid == _NS - 1)
        def _():
            pltpu.sync_copy(
                acc.at[pl.ds(_NS * _RPS, _TAIL)],
                out_hbm.at[pl.ds(cid * _N + _NS * _RPS, _TAIL)])

    return seg_sum(m_a, m_b, src3d, dst3d, zrows)


def _dot(a, b):
    return jnp.dot(a, b, preferred_element_type=jnp.float32)


def _proj_body(x_ref, wp_ref, bp_ref, wga_ref, wgb_ref, h_ref, ma_ref,
               mb_ref):
    h = jnp.maximum(_dot(x_ref[...], wp_ref[...]) + bp_ref[...], 0.0)
    h_ref[...] = h
    ma_ref[...] = _dot(h, wga_ref[...])
    mb_ref[...] = _dot(h, wgb_ref[...])


def _proj(x, wpT, bp, wgA, wgB):
    return pl.pallas_call(
        _proj_body,
        grid=(_NBLK,),
        in_specs=[
            pl.BlockSpec((_BLK, _IN), lambda i: (i, 0)),
            pl.BlockSpec((_IN, _H), lambda i: (0, 0)),
            pl.BlockSpec((1, _H), lambda i: (0, 0)),
            pl.BlockSpec((_H, _HA), lambda i: (0, 0)),
            pl.BlockSpec((_H, _HA), lambda i: (0, 0)),
        ],
        out_specs=[
            pl.BlockSpec((_BLK, _H), lambda i: (i, 0)),
            pl.BlockSpec((_BLK, _HA), lambda i: (i, 0)),
            pl.BlockSpec((_BLK, _HA), lambda i: (i, 0)),
        ],
        out_shape=[
            jax.ShapeDtypeStruct((_N, _H), jnp.float32),
            jax.ShapeDtypeStruct((_N, _HA), jnp.float32),
            jax.ShapeDtypeStruct((_N, _HA), jnp.float32),
        ],
    )(x, wpT, bp, wgA, wgB)


def _gru_compute(aa, ab, h, wm, bias):
    (wira, wirb, wiza, wizb, wina, winb, whr, whz, whn) = wm
    r = jax.nn.sigmoid(_dot(aa, wira) + _dot(ab, wirb) + _dot(h, whr)
                       + bias[0:1, :])
    z = jax.nn.sigmoid(_dot(aa, wiza) + _dot(ab, wizb) + _dot(h, whz)
                       + bias[1:2, :])
    n = jnp.tanh(_dot(aa, wina) + _dot(ab, winb) + bias[2:3, :]
                 + r * (_dot(h, whn) + bias[3:4, :]))
    return (1.0 - z) * n + z * h


def _gru_next_body(aa_ref, ab_ref, h_ref, w0, w1, w2, w3, w4, w5, w6, w7, w8,
                   bias_ref, wga_ref, wgb_ref, hn_ref, ma_ref, mb_ref):
    wm = tuple(w[...] for w in (w0, w1, w2, w3, w4, w5, w6, w7, w8))
    h_new = _gru_compute(aa_ref[...], ab_ref[...], h_ref[...], wm, bias_ref[...])
    hn_ref[...] = h_new
    ma_ref[...] = _dot(h_new, wga_ref[...])
    mb_ref[...] = _dot(h_new, wgb_ref[...])


def _gru_last_body(aa_ref, ab_ref, h_ref, w0, w1, w2, w3, w4, w5, w6, w7, w8,
                   bias_ref, hn_ref):
    wm = tuple(w[...] for w in (w0, w1, w2, w3, w4, w5, w6, w7, w8))
    hn_ref[...] = _gru_compute(aa_ref[...], ab_ref[...], h_ref[...], wm,
                               bias_ref[...])


def _gru_step(parts, h, wmats, bias, wg_next):
    wspecs = ([pl.BlockSpec((_HA, _H), lambda i: (0, 0))] * 6
              + [pl.BlockSpec((_H, _H), lambda i: (0, 0))] * 3)
    in_specs = [
        pl.BlockSpec((_BLK, _HA), lambda i: (i, 0)),
        pl.BlockSpec((_BLK, _HA), lambda i: (i + _NBLK, 0)),
        pl.BlockSpec((_BLK, _H), lambda i: (i, 0)),
    ] + wspecs + [pl.BlockSpec((8, _H), lambda i: (0, 0))]
    if wg_next is not None:
        return pl.pallas_call(
            _gru_next_body,
            grid=(_NBLK,),
            in_specs=in_specs + [
                pl.BlockSpec((_H, _HA), lambda i: (0, 0)),
                pl.BlockSpec((_H, _HA), lambda i: (0, 0)),
            ],
            out_specs=[
                pl.BlockSpec((_BLK, _H), lambda i: (i, 0)),
                pl.BlockSpec((_BLK, _HA), lambda i: (i, 0)),
                pl.BlockSpec((_BLK, _HA), lambda i: (i, 0)),
            ],
            out_shape=[
                jax.ShapeDtypeStruct((_N, _H), jnp.float32),
                jax.ShapeDtypeStruct((_N, _HA), jnp.float32),
                jax.ShapeDtypeStruct((_N, _HA), jnp.float32),
            ],
        )(parts, parts, h, *wmats, bias, *wg_next)
    return pl.pallas_call(
        _gru_last_body,
        grid=(_NBLK,),
        in_specs=in_specs,
        out_specs=pl.BlockSpec((_BLK, _H), lambda i: (i, 0)),
        out_shape=jax.ShapeDtypeStruct((_N, _H), jnp.float32),
    )(parts, parts, h, *wmats, bias)


def _pool_mlp_body(h_ref, b_ref, w1_ref, b1_ref, w2_ref, b2_ref, out_ref,
                   acc_ref):
    i = pl.program_id(0)
    hb = h_ref[...]
    bb = b_ref[...]
    neg = jnp.float32(-jnp.inf)
    local = jnp.stack(
        [jnp.max(jnp.where(bb == g, hb, neg), axis=0) for g in range(_G)])

    @pl.when(i == 0)
    def _():
        acc_ref[...] = local

    @pl.when(i > 0)
    def _():
        acc_ref[...] = jnp.maximum(acc_ref[...], local)

    @pl.when(i == _NBLK - 1)
    def _():
        pooled = jnp.maximum(acc_ref[...], 0.0)
        hid = jnp.maximum(_dot(pooled, w1_ref[...]) + b1_ref[...], 0.0)
        out_ref[...] = _dot(hid, w2_ref[...]) + b2_ref[...]


def _pool_mlp(h, batch2d, w1T, b1, w2T, b2):
    return pl.pallas_call(
        _pool_mlp_body,
        grid=(_NBLK,),
        in_specs=[
            pl.BlockSpec((_BLK, _H), lambda i: (i, 0)),
            pl.BlockSpec((_BLK, 1), lambda i: (i, 0)),
            pl.BlockSpec((_H, _HID), lambda i: (0, 0)),
            pl.BlockSpec((1, _HID), lambda i: (0, 0)),
            pl.BlockSpec((_HID, _OUT), lambda i: (0, 0)),
            pl.BlockSpec((1, _OUT), lambda i: (0, 0)),
        ],
        out_specs=pl.BlockSpec((_G, _OUT), lambda i: (0, 0)),
        out_shape=jax.ShapeDtypeStruct((_G, _OUT), jnp.float32),
        scratch_shapes=[pltpu.VMEM((_G, _H), jnp.float32)],
    )(h, batch2d, w1T, b1, w2T, b2)


def _split_k(wT):
    top = wT[:_HA, :]
    bot = jnp.concatenate(
        [wT[_HA:, :], jnp.zeros((_HA - _HB, _H), jnp.float32)], axis=0)
    return top, bot


def _split_wg(wg):
    wga = wg[:, :_HA]
    wgb = jnp.concatenate(
        [wg[:, _HA:], jnp.zeros((_H, _HA - _HB), jnp.float32)], axis=1)
    return wga, wgb


def kernel(x, edge_index, batch, W_proj, b_proj, Wg, W_ih, b_ih, W_hh, b_hh,
           W1, b1, W2, b2):
    src3d = edge_index[0].reshape(_NS, _GROUPS, _GITERS, _CH)
    dst3d = edge_index[1].reshape(_NS, _GROUPS, _GITERS, _CH)
    zrows = jnp.zeros((_RPS, _HA), jnp.float32)

    wi = jnp.split(W_ih, 3, axis=0)
    wh = jnp.split(W_hh, 3, axis=0)
    wmats = (_split_k(wi[0].T) + _split_k(wi[1].T) + _split_k(wi[2].T)
             + (wh[0].T, wh[1].T, wh[2].T))
    bi = jnp.split(b_ih, 3)
    bh = jnp.split(b_hh, 3)
    bias = jnp.concatenate(
        [jnp.stack([bi[0] + bh[0], bi[1] + bh[1], bi[2], bh[2]]),
         jnp.zeros((4, _H), jnp.float32)])

    wg0a, wg0b = _split_wg(Wg[0])
    h, m_a, m_b = _proj(x, W_proj.T, b_proj[None, :], wg0a, wg0b)
    for i in range(_STEPS):
        parts = _sc_segment_sum(m_a, m_b, src3d, dst3d, zrows)
        if i + 1 < _STEPS:
            h, m_a, m_b = _gru_step(parts, h, wmats, bias, _split_wg(Wg[i + 1]))
        else:
            h = _gru_step(parts, h, wmats, bias, None)

    return _pool_mlp(h, batch[:, None], W1.T, b1[None, :], W2.T, b2[None, :])


# speedup vs baseline: 6.0950x; 1.2540x over previous
import functools

import jax
import jax.numpy as jnp
from jax import lax
from jax.experimental import pallas as pl
from jax.experimental.pallas import tpu as pltpu
from jax.experimental.pallas import tpu_sc as plsc

_N = 10000
_E = 320000
_IN = 128
_H = 200
_HA = 128
_HB = _H - _HA
_STEPS = 8
_G = 64
_HID = 100
_OUT = 2

_NC = 2
_NS = 16
_CH = 80
_EPT = _E // _NS
_ITERS = _EPT // _CH
_GITERS = 10
_GROUPS = _ITERS // _GITERS
_RPS = 624
_TAIL = _N - _NS * _RPS

_BLK = 1000
_NBLK = _N // _BLK


def _sc_segment_sum(m_a, m_b, src3d, dst3d, zrows):
    mesh = plsc.VectorSubcoreMesh(core_axis_name="c", subcore_axis_name="s")

    @functools.partial(
        pl.kernel,
        mesh=mesh,
        out_type=jax.ShapeDtypeStruct((_NC * _N, _HA), jnp.float32),
        scratch_types=[
            pltpu.VMEM((1, 1, _GITERS, _CH), jnp.int32),
            pltpu.VMEM((1, 1, _GITERS, _CH), jnp.int32),
            pltpu.VMEM((4, _CH, _HA), jnp.float32),
            pltpu.VMEM_SHARED((_N, _HA), jnp.float32),
            pltpu.SemaphoreType.DMA,
            pltpu.SemaphoreType.DMA,
            pltpu.SemaphoreType.DMA,
            pltpu.SemaphoreType.DMA,
            pltpu.SemaphoreType.DMA,
            pltpu.SemaphoreType.DMA,
            pltpu.SemaphoreType.DMA,
            pltpu.SemaphoreType.DMA,
        ],
    )
    def seg_sum(ma_hbm, mb_hbm, src_hbm, dst_hbm, z_hbm, out_hbm, sidx, didx,
                rows, acc, g0, g1, g2, g3, s0, s1, s2, s3):
        cid = lax.axis_index("c")
        sid = lax.axis_index("s")
        pltpu.sync_copy(z_hbm.at[pl.ds(0, _RPS)],
                        acc.at[pl.ds(sid * _RPS, _RPS)])

        @pl.when(sid == _NS - 1)
        def _():
            pltpu.sync_copy(z_hbm.at[pl.ds(0, _TAIL)],
                            acc.at[pl.ds(_NS * _RPS, _TAIL)])

        plsc.subcore_barrier()

        gsems = (g0, g1, g2, g3)
        ssems = (s0, s1, s2, s3)

        def _accumulate(m_hbm):
            def group(g, carry):
                pltpu.sync_copy(src_hbm.at[pl.ds(sid, 1), pl.ds(g, 1)], sidx)
                pltpu.sync_copy(dst_hbm.at[pl.ds(sid, 1), pl.ds(g, 1)], didx)
                gd = [None] * _GITERS
                sd = [None] * _GITERS
                for j in range(_GITERS):
                    b = j % 4
                    if j >= 4:
                        sd[j - 4].wait()
                    gd[j] = pltpu.async_copy(
                        m_hbm.at[sidx.at[0, 0, j]], rows.at[b], gsems[b])
                    if j >= 1:
                        gd[j - 1].wait()
                        sd[j - 1] = pltpu.async_copy(
                            rows.at[(j - 1) % 4],
                            acc.at[didx.at[0, 0, j - 1]],
                            ssems[(j - 1) % 4], add=True)
                last = _GITERS - 1
                gd[last].wait()
                sd[last] = pltpu.async_copy(
                    rows.at[last % 4], acc.at[didx.at[0, 0, last]],
                    ssems[last % 4], add=True)
                for j in range(_GITERS - 4, _GITERS):
                    sd[j].wait()
                return carry

            lax.fori_loop(0, _GROUPS, group, 0)

        @pl.when(cid == 0)
        def _():
            _accumulate(ma_hbm)

        @pl.when(cid == 1)
        def _():
            _accumulate(mb_hbm)

        plsc.subcore_barrier()
        pltpu.sync_copy(
            acc.at[pl.ds(sid * _RPS, _RPS)],
            out_hbm.at[pl.ds(cid * _N + sid * _RPS, _RPS)])

        @pl.when(sid == _NS - 1)
        def _():
            pltpu.sync_copy(
                acc.at[pl.ds(_NS * _RPS, _TAIL)],
                out_hbm.at[pl.ds(cid * _N + _NS * _RPS, _TAIL)])

    return seg_sum(m_a, m_b, src3d, dst3d, zrows)


def _dot(a, b):
    return jnp.dot(a, b, preferred_element_type=jnp.float32)


def _proj_body(x_ref, wp_ref, bp_ref, wga_ref, wgb_ref, h_ref, ma_ref,
               mb_ref):
    h = jnp.maximum(_dot(x_ref[...], wp_ref[...]) + bp_ref[...], 0.0)
    h_ref[...] = h
    ma_ref[...] = _dot(h, wga_ref[...])
    mb_ref[...] = _dot(h, wgb_ref[...])


def _proj(x, wpT, bp, wgA, wgB):
    return pl.pallas_call(
        _proj_body,
        grid=(_NBLK,),
        in_specs=[
            pl.BlockSpec((_BLK, _IN), lambda i: (i, 0)),
            pl.BlockSpec((_IN, _H), lambda i: (0, 0)),
            pl.BlockSpec((1, _H), lambda i: (0, 0)),
            pl.BlockSpec((_H, _HA), lambda i: (0, 0)),
            pl.BlockSpec((_H, _HA), lambda i: (0, 0)),
        ],
        out_specs=[
            pl.BlockSpec((_BLK, _H), lambda i: (i, 0)),
            pl.BlockSpec((_BLK, _HA), lambda i: (i, 0)),
            pl.BlockSpec((_BLK, _HA), lambda i: (i, 0)),
        ],
        out_shape=[
            jax.ShapeDtypeStruct((_N, _H), jnp.float32),
            jax.ShapeDtypeStruct((_N, _HA), jnp.float32),
            jax.ShapeDtypeStruct((_N, _HA), jnp.float32),
        ],
    )(x, wpT, bp, wgA, wgB)


def _gru_compute(aa, ab, h, wm, bias):
    (wira, wirb, wiza, wizb, wina, winb, whr, whz, whn) = wm
    r = jax.nn.sigmoid(_dot(aa, wira) + _dot(ab, wirb) + _dot(h, whr)
                       + bias[0:1, :])
    z = jax.nn.sigmoid(_dot(aa, wiza) + _dot(ab, wizb) + _dot(h, whz)
                       + bias[1:2, :])
    n = jnp.tanh(_dot(aa, wina) + _dot(ab, winb) + bias[2:3, :]
                 + r * (_dot(h, whn) + bias[3:4, :]))
    return (1.0 - z) * n + z * h


def _gru_next_body(aa_ref, ab_ref, h_ref, w0, w1, w2, w3, w4, w5, w6, w7, w8,
                   bias_ref, wga_ref, wgb_ref, hn_ref, ma_ref, mb_ref):
    wm = tuple(w[...] for w in (w0, w1, w2, w3, w4, w5, w6, w7, w8))
    h_new = _gru_compute(aa_ref[...], ab_ref[...], h_ref[...], wm, bias_ref[...])
    hn_ref[...] = h_new
    ma_ref[...] = _dot(h_new, wga_ref[...])
    mb_ref[...] = _dot(h_new, wgb_ref[...])


def _gru_last_body(aa_ref, ab_ref, h_ref, w0, w1, w2, w3, w4, w5, w6, w7, w8,
                   bias_ref, hn_ref):
    wm = tuple(w[...] for w in (w0, w1, w2, w3, w4, w5, w6, w7, w8))
    hn_ref[...] = _gru_compute(aa_ref[...], ab_ref[...], h_ref[...], wm,
                               bias_ref[...])


def _gru_step(parts, h, wmats, bias, wg_next):
    wspecs = ([pl.BlockSpec((_HA, _H), lambda i: (0, 0))] * 6
              + [pl.BlockSpec((_H, _H), lambda i: (0, 0))] * 3)
    in_specs = [
        pl.BlockSpec((_BLK, _HA), lambda i: (i, 0)),
        pl.BlockSpec((_BLK, _HA), lambda i: (i + _NBLK, 0)),
        pl.BlockSpec((_BLK, _H), lambda i: (i, 0)),
    ] + wspecs + [pl.BlockSpec((8, _H), lambda i: (0, 0))]
    if wg_next is not None:
        return pl.pallas_call(
            _gru_next_body,
            grid=(_NBLK,),
            in_specs=in_specs + [
                pl.BlockSpec((_H, _HA), lambda i: (0, 0)),
                pl.BlockSpec((_H, _HA), lambda i: (0, 0)),
            ],
            out_specs=[
                pl.BlockSpec((_BLK, _H), lambda i: (i, 0)),
                pl.BlockSpec((_BLK, _HA), lambda i: (i, 0)),
                pl.BlockSpec((_BLK, _HA), lambda i: (i, 0)),
            ],
            out_shape=[
                jax.ShapeDtypeStruct((_N, _H), jnp.float32),
                jax.ShapeDtypeStruct((_N, _HA), jnp.float32),
                jax.ShapeDtypeStruct((_N, _HA), jnp.float32),
            ],
        )(parts, parts, h, *wmats, bias, *wg_next)
    return pl.pallas_call(
        _gru_last_body,
        grid=(_NBLK,),
        in_specs=in_specs,
        out_specs=pl.BlockSpec((_BLK, _H), lambda i: (i, 0)),
        out_shape=jax.ShapeDtypeStruct((_N, _H), jnp.float32),
    )(parts, parts, h, *wmats, bias)


def _pool_mlp_body(h_ref, b_ref, w1_ref, b1_ref, w2_ref, b2_ref, out_ref,
                   acc_ref):
    i = pl.program_id(0)
    hb = h_ref[...]
    bb = b_ref[...]
    neg = jnp.float32(-jnp.inf)
    local = jnp.stack(
        [jnp.max(jnp.where(bb == g, hb, neg), axis=0) for g in range(_G)])

    @pl.when(i == 0)
    def _():
        acc_ref[...] = local

    @pl.when(i > 0)
    def _():
        acc_ref[...] = jnp.maximum(acc_ref[...], local)

    @pl.when(i == _NBLK - 1)
    def _():
        pooled = jnp.maximum(acc_ref[...], 0.0)
        hid = jnp.maximum(_dot(pooled, w1_ref[...]) + b1_ref[...], 0.0)
        out_ref[...] = _dot(hid, w2_ref[...]) + b2_ref[...]


def _pool_mlp(h, batch2d, w1T, b1, w2T, b2):
    return pl.pallas_call(
        _pool_mlp_body,
        grid=(_NBLK,),
        in_specs=[
            pl.BlockSpec((_BLK, _H), lambda i: (i, 0)),
            pl.BlockSpec((_BLK, 1), lambda i: (i, 0)),
            pl.BlockSpec((_H, _HID), lambda i: (0, 0)),
            pl.BlockSpec((1, _HID), lambda i: (0, 0)),
            pl.BlockSpec((_HID, _OUT), lambda i: (0, 0)),
            pl.BlockSpec((1, _OUT), lambda i: (0, 0)),
        ],
        out_specs=pl.BlockSpec((_G, _OUT), lambda i: (0, 0)),
        out_shape=jax.ShapeDtypeStruct((_G, _OUT), jnp.float32),
        scratch_shapes=[pltpu.VMEM((_G, _H), jnp.float32)],
    )(h, batch2d, w1T, b1, w2T, b2)


def _split_k(wT):
    top = wT[:_HA, :]
    bot = jnp.concatenate(
        [wT[_HA:, :], jnp.zeros((_HA - _HB, _H), jnp.float32)], axis=0)
    return top, bot


def _split_wg(wg):
    wga = wg[:, :_HA]
    wgb = jnp.concatenate(
        [wg[:, _HA:], jnp.zeros((_H, _HA - _HB), jnp.float32)], axis=1)
    return wga, wgb


def kernel(x, edge_index, batch, W_proj, b_proj, Wg, W_ih, b_ih, W_hh, b_hh,
           W1, b1, W2, b2):
    src3d = edge_index[0].reshape(_NS, _GROUPS, _GITERS, _CH)
    dst3d = edge_index[1].reshape(_NS, _GROUPS, _GITERS, _CH)
    zrows = jnp.zeros((_RPS, _HA), jnp.float32)

    wi = jnp.split(W_ih, 3, axis=0)
    wh = jnp.split(W_hh, 3, axis=0)
    wmats = (_split_k(wi[0].T) + _split_k(wi[1].T) + _split_k(wi[2].T)
             + (wh[0].T, wh[1].T, wh[2].T))
    bi = jnp.split(b_ih, 3)
    bh = jnp.split(b_hh, 3)
    bias = jnp.concatenate(
        [jnp.stack([bi[0] + bh[0], bi[1] + bh[1], bi[2], bh[2]]),
         jnp.zeros((4, _H), jnp.float32)])

    wg0a, wg0b = _split_wg(Wg[0])
    h, m_a, m_b = _proj(x, W_proj.T, b_proj[None, :], wg0a, wg0b)
    for i in range(_STEPS):
        parts = _sc_segment_sum(m_a, m_b, src3d, dst3d, zrows)
        if i + 1 < _STEPS:
            h, m_a, m_b = _gru_step(parts, h, wmats, bias, _split_wg(Wg[i + 1]))
        else:
            h = _gru_step(parts, h, wmats, bias, None)

    return _pool_mlp(h, batch[:, None], W1.T, b1[None, :], W2.T, b2[None, :])
